# baseline (speedup 1.0000x reference)
import functools

import jax
import jax.numpy as jnp
from jax import lax
from jax.experimental import pallas as pl
from jax.experimental.pallas import tpu as pltpu
from jax.experimental.pallas import tpu_sc as plsc

DIM = 64
NUM_CORES = 2
NUM_SUBCORES = 16
NW = NUM_CORES * NUM_SUBCORES

CHUNK = 256
NBUF = 4

BATCH = 16384
HIST = 50


def _make_lookup(n):
    b_per_w = n // NW
    n_chunks = b_per_w // CHUNK
    t_steady = n_chunks // NBUF - 1
    mesh = plsc.VectorSubcoreMesh(core_axis_name="c", subcore_axis_name="s")

    @functools.partial(
        pl.kernel,
        mesh=mesh,
        out_type=jax.ShapeDtypeStruct((n, DIM), jnp.float32),
        scratch_types=[
            pltpu.VMEM((b_per_w,), jnp.int32),
            [pltpu.VMEM((CHUNK, DIM), jnp.float32) for _ in range(NBUF)],
            [pltpu.SemaphoreType.DMA for _ in range(NBUF)],
            [pltpu.SemaphoreType.DMA for _ in range(NBUF)],
        ],
        compiler_params=pltpu.CompilerParams(use_tc_tiling_on_sc=False),
    )
    def lookup(idx_hbm, table_hbm, out_hbm, idx_v, rows, gsem, wsem):
        wid = lax.axis_index("s") * NUM_CORES + lax.axis_index("c")
        base = wid * b_per_w
        pltpu.sync_copy(idx_hbm.at[pl.ds(base, b_per_w)], idx_v)

        def start_gather(c, b):
            pltpu.async_copy(
                table_hbm.at[idx_v.at[pl.ds(c * CHUNK, CHUNK)]], rows[b], gsem[b]
            )

        def wait_gather(b):
            pltpu.make_async_copy(
                table_hbm.at[idx_v.at[pl.ds(0, CHUNK)]], rows[b], gsem[b]
            ).wait()

        def start_write(c, b):
            pltpu.async_copy(
                rows[b], out_hbm.at[pl.ds(base + c * CHUNK, CHUNK)], wsem[b]
            )

        def wait_write(b):
            pltpu.make_async_copy(
                rows[b], out_hbm.at[pl.ds(base, CHUNK)], wsem[b]
            ).wait()

        for b in range(NBUF):
            start_gather(b, b)

        def body(t, _):
            c0 = t * NBUF
            for b in range(NBUF):
                wait_gather(b)
                start_write(c0 + b, b)
            for b in range(NBUF):
                wait_write(b)
                start_gather(c0 + NBUF + b, b)
            return ()

        lax.fori_loop(0, t_steady, body, (), unroll=False)

        c0 = t_steady * NBUF
        for b in range(NBUF):
            wait_gather(b)
            start_write(c0 + b, b)
        for b in range(NBUF):
            wait_write(b)

    return lookup


def _tc_transpose_kernel(rows_hbm, out_hbm, ibuf, obuf, isem, osem):
    def start_in(h):
        pltpu.make_async_copy(
            rows_hbm.at[pl.ds(h * BATCH, BATCH)], ibuf[h % 2], isem[h % 2]
        ).start()

    def wait_in(h):
        pltpu.make_async_copy(
            rows_hbm.at[pl.ds(h * BATCH, BATCH)], ibuf[h % 2], isem[h % 2]
        ).wait()

    def start_out(h):
        pltpu.make_async_copy(obuf[h % 2], out_hbm.at[h], osem[h % 2]).start()

    def wait_out(h):
        pltpu.make_async_copy(obuf[h % 2], out_hbm.at[h], osem[h % 2]).wait()

    start_in(0)
    for h in range(HIST):
        if h + 1 < HIST:
            start_in(h + 1)
        wait_in(h)
        if h >= 2:
            wait_out(h - 2)
        obuf[h % 2][...] = ibuf[h % 2][...].T
        start_out(h)
    wait_out(HIST - 2)
    wait_out(HIST - 1)


def _tc_transpose(rows):
    return pl.pallas_call(
        _tc_transpose_kernel,
        out_shape=jax.ShapeDtypeStruct((HIST, DIM, BATCH), jnp.float32),
        in_specs=[pl.BlockSpec(memory_space=pl.ANY)],
        out_specs=pl.BlockSpec(memory_space=pl.ANY),
        scratch_shapes=[
            [pltpu.VMEM((BATCH, DIM), jnp.float32) for _ in range(2)],
            [pltpu.VMEM((DIM, BATCH), jnp.float32) for _ in range(2)],
            [pltpu.SemaphoreType.DMA for _ in range(2)],
            [pltpu.SemaphoreType.DMA for _ in range(2)],
        ],
    )(rows)


def kernel(x, table):
    batch, hist = x.shape
    n = batch * hist
    idx = x.T.reshape(n).astype(jnp.int32)
    rows = _make_lookup(n)(idx, table)
    out_t = _tc_transpose(rows)
    return out_t.transpose(2, 0, 1)


# speedup vs baseline: 1.4159x; 1.4159x over previous
import functools

import jax
import jax.numpy as jnp
from jax import lax
from jax.experimental import pallas as pl
from jax.experimental.pallas import tpu as pltpu
from jax.experimental.pallas import tpu_sc as plsc

DIM = 64
NUM_CORES = 2
NUM_SUBCORES = 16
NW = NUM_CORES * NUM_SUBCORES

BATCH = 16384
HIST = 50
COLS = BATCH // NW
NBUF = 2


def _make_lookup():
    t_steady = HIST // NBUF - 1
    mesh = plsc.VectorSubcoreMesh(core_axis_name="c", subcore_axis_name="s")

    @functools.partial(
        pl.kernel,
        mesh=mesh,
        out_type=jax.ShapeDtypeStruct((HIST, BATCH // 2, 2 * DIM), jnp.float32),
        scratch_types=[
            pltpu.VMEM((HIST, COLS), jnp.int32),
            [pltpu.VMEM((COLS, DIM), jnp.float32) for _ in range(NBUF)],
            [pltpu.SemaphoreType.DMA for _ in range(NBUF)],
            [pltpu.SemaphoreType.DMA for _ in range(NBUF)],
        ],
        compiler_params=pltpu.CompilerParams(use_tc_tiling_on_sc=False),
    )
    def lookup(idx_hbm, table_hbm, out_hbm, idx_v, rows, gsem, wsem):
        wid = lax.axis_index("s") * NUM_CORES + lax.axis_index("c")
        col0 = wid * COLS
        rowoff = lax.rem(col0, BATCH // 2)
        coloff = lax.div(col0, BATCH // 2) * DIM
        pltpu.sync_copy(idx_hbm.at[:, pl.ds(col0, COLS)], idx_v)

        def start_gather(h, b):
            pltpu.async_copy(
                table_hbm.at[idx_v.at[h]], rows[b], gsem[b]
            )

        def wait_gather(b):
            pltpu.make_async_copy(
                table_hbm.at[idx_v.at[0]], rows[b], gsem[b]
            ).wait()

        def start_write(h, b):
            dst = out_hbm.at[h, pl.ds(rowoff, COLS), pl.ds(coloff, DIM)]
            pltpu.async_copy(rows[b], dst, wsem[b])

        def wait_write(b):
            dst = out_hbm.at[0, pl.ds(rowoff, COLS), pl.ds(coloff, DIM)]
            pltpu.make_async_copy(rows[b], dst, wsem[b]).wait()

        for b in range(NBUF):
            start_gather(b, b)

        def body(t, _):
            h0 = t * NBUF
            for b in range(NBUF):
                wait_gather(b)
                start_write(h0 + b, b)
            for b in range(NBUF):
                wait_write(b)
                start_gather(h0 + NBUF + b, b)
            return ()

        lax.fori_loop(0, t_steady, body, (), unroll=False)

        h0 = t_steady * NBUF
        for b in range(NBUF):
            wait_gather(b)
            start_write(h0 + b, b)
        for b in range(NBUF):
            wait_write(b)

    return lookup


def _tc_transpose_kernel(rows_hbm, out_hbm, ibuf, obuf, isem, osem):
    H2 = BATCH // 2

    def start_in(h):
        pltpu.make_async_copy(rows_hbm.at[h], ibuf[h % 2], isem[h % 2]).start()

    def wait_in(h):
        pltpu.make_async_copy(rows_hbm.at[h], ibuf[h % 2], isem[h % 2]).wait()

    def _out_copies(h):
        b = h % 2
        return (
            pltpu.make_async_copy(
                obuf[b].at[pl.ds(0, DIM)], out_hbm.at[h, :, pl.ds(0, H2)], osem[2 * b]
            ),
            pltpu.make_async_copy(
                obuf[b].at[pl.ds(DIM, DIM)],
                out_hbm.at[h, :, pl.ds(H2, H2)],
                osem[2 * b + 1],
            ),
        )

    def start_out(h):
        for c in _out_copies(h):
            c.start()

    def wait_out(h):
        for c in _out_copies(h):
            c.wait()

    start_in(0)
    for h in range(HIST):
        if h + 1 < HIST:
            start_in(h + 1)
        wait_in(h)
        if h >= 2:
            wait_out(h - 2)
        obuf[h % 2][...] = ibuf[h % 2][...].T
        start_out(h)
    wait_out(HIST - 2)
    wait_out(HIST - 1)


def _tc_transpose(rows):
    return pl.pallas_call(
        _tc_transpose_kernel,
        out_shape=jax.ShapeDtypeStruct((HIST, DIM, BATCH), jnp.float32),
        in_specs=[pl.BlockSpec(memory_space=pl.ANY)],
        out_specs=pl.BlockSpec(memory_space=pl.ANY),
        scratch_shapes=[
            [pltpu.VMEM((BATCH // 2, 2 * DIM), jnp.float32) for _ in range(2)],
            [pltpu.VMEM((2 * DIM, BATCH // 2), jnp.float32) for _ in range(2)],
            [pltpu.SemaphoreType.DMA for _ in range(2)],
            [pltpu.SemaphoreType.DMA for _ in range(4)],
        ],
    )(rows)


def kernel(x, table):
    idx = x.T.astype(jnp.int32)
    rows = _make_lookup()(idx, table)
    out_t = _tc_transpose(rows)
    return out_t.transpose(2, 0, 1)
